# baseline (speedup 1.0000x reference)
import functools

import jax
import jax.numpy as jnp
from jax import lax
from jax.experimental import pallas as pl
from jax.experimental.pallas import tpu as pltpu
from jax.experimental.pallas import tpu_sc as plsc


def _deg_kernel(E, N):
    EPT = E // 32
    HALF = N // 2
    HPAD = HALF + 8
    NV = EPT // 16

    mesh = plsc.VectorSubcoreMesh(core_axis_name="c", subcore_axis_name="s")

    @functools.partial(
        pl.kernel,
        out_type=jax.ShapeDtypeStruct((32 * N,), jnp.float32),
        mesh=mesh,
        compiler_params=pltpu.CompilerParams(needs_layout_passes=False),
        scratch_types=[
            pltpu.VMEM((EPT,), jnp.int32),
            pltpu.VMEM((16 * HPAD,), jnp.float32),
            pltpu.VMEM((HPAD,), jnp.float32),
        ],
    )
    def deg(dst_hbm, out_hbm, dst_v, hist_v, acc_v):
        cid = lax.axis_index("c")
        sid = lax.axis_index("s")
        wid = sid * 2 + cid
        pltpu.sync_copy(dst_hbm.at[pl.ds(wid * EPT, EPT)], dst_v)
        lane_base = lax.iota(jnp.int32, 16) * HPAD
        ones = jnp.ones((16,), jnp.float32)
        zeros16 = jnp.zeros((16,), jnp.float32)
        for h in range(2):
            lo = h * HALF

            def zbody(j, _):
                hist_v[pl.ds(j * 16, 16)] = zeros16
                return 0
            lax.fori_loop(0, 16 * HPAD // 16, zbody, 0)

            def sbody(j, _):
                d = dst_v[pl.ds(j * 16, 16)]
                m = (d >= lo) & (d < lo + HALF)
                idx = lane_base + (d - lo)
                cur = plsc.load_gather(hist_v, [idx], mask=m)
                plsc.store_scatter(hist_v, [idx], cur + ones, mask=m)
                return 0
            lax.fori_loop(0, NV, sbody, 0)

            def rbody(j, _):
                a = hist_v[pl.ds(j * 16, 16)]
                for l in range(1, 16):
                    a = a + hist_v[pl.ds(l * HPAD + j * 16, 16)]
                acc_v[pl.ds(j * 16, 16)] = a
                return 0
            lax.fori_loop(0, HPAD // 16, rbody, 0)
            pltpu.sync_copy(acc_v.at[pl.ds(0, HALF)],
                            out_hbm.at[pl.ds(wid * N + lo, HALF)])

    return deg


def _proj_kernel(B, N, F, P, OUT, BN):
    grid = (B, P // 2, N // BN)
    NB = N // BN

    def body(x_ref, wz_ref, lwz_ref, wh_ref, lwh_ref, degp_ref, u_ref):
        wc = jnp.concatenate(
            [jnp.dot(wz_ref[...], lwz_ref[:OUT, :],
                     preferred_element_type=jnp.float32),
             jnp.dot(wh_ref[...], lwh_ref[:OUT, :],
                     preferred_element_type=jnp.float32)], axis=1)
        deg = jnp.sum(degp_ref[...], axis=1) + 1.0
        dinv = lax.rsqrt(deg)
        cols = []
        for tt in range(2):
            xt = x_ref[0, tt]
            m = jnp.dot(xt, wc, preferred_element_type=jnp.float32)
            cols.append(m * dinv[:, None])
        u_ref[...] = jnp.concatenate(cols, axis=1)

    return pl.pallas_call(
        body,
        grid=grid,
        in_specs=[
            pl.BlockSpec((1, 2, BN, F), lambda b, tp, nb: (b, tp, nb, 0)),
            pl.BlockSpec((F, OUT), lambda b, tp, nb: (0, 0)),
            pl.BlockSpec((2 * OUT, OUT), lambda b, tp, nb: (0, 0)),
            pl.BlockSpec((F, OUT), lambda b, tp, nb: (0, 0)),
            pl.BlockSpec((2 * OUT, OUT), lambda b, tp, nb: (0, 0)),
            pl.BlockSpec((BN, 32), lambda b, tp, nb: (nb, 0)),
        ],
        out_specs=pl.BlockSpec(
            (BN, 128), lambda b, tp, nb: ((b * (P // 2) + tp) * NB + nb, 0)),
        out_shape=jax.ShapeDtypeStruct((2 * P * N, 128), jnp.float32),
    )


def _agg_kernel(E, N, NSLICE):
    EPT = E // 16
    CH = 4096
    CE = 4000
    NCH = EPT // CE
    G = 128
    SINK = N
    RPT = 624

    mesh = plsc.VectorSubcoreMesh(core_axis_name="c", subcore_axis_name="s")

    @functools.partial(
        pl.kernel,
        out_type=jax.ShapeDtypeStruct((NSLICE * N, 128), jnp.float32),
        mesh=mesh,
        compiler_params=pltpu.CompilerParams(needs_layout_passes=False),
        scratch_types=[
            pltpu.VMEM((CH,), jnp.int32),
            pltpu.VMEM((CH,), jnp.int32),
            pltpu.VMEM((G,), jnp.int32),
            pltpu.VMEM((G,), jnp.int32),
            pltpu.VMEM((G,), jnp.int32),
            pltpu.VMEM((G,), jnp.int32),
            pltpu.VMEM((G, 128), jnp.float32),
            pltpu.VMEM((G, 128), jnp.float32),
            pltpu.VMEM_SHARED((N + 8, 128), jnp.float32),
            pltpu.SemaphoreType.DMA,
            pltpu.SemaphoreType.DMA,
            pltpu.SemaphoreType.DMA,
            pltpu.SemaphoreType.DMA,
        ],
    )
    def agg(u_hbm, esrc_hbm, edst_hbm, y_hbm,
            esrc_v, edst_v, gb0_v, db0_v, gb1_v, db1_v, rb0, rb1, acc,
            sg0, sg1, ss0, ss1):
        cid = lax.axis_index("c")
        sid = lax.axis_index("s")
        zero16 = jnp.zeros((16,), jnp.int32)
        sink16 = jnp.full((16,), SINK, jnp.int32)
        for q in range((CH - CE) // 16):
            esrc_v[pl.ds(CE + q * 16, 16)] = zero16
            edst_v[pl.ds(CE + q * 16, 16)] = sink16

        def slice_body(s_i, _):
            s = cid * (NSLICE // 2) + s_i
            ubase = pl.multiple_of(s * N, 8)
            pltpu.sync_copy(u_hbm.at[pl.ds(ubase + sid * RPT, RPT)],
                            acc.at[pl.ds(sid * RPT, RPT)])

            @pl.when(sid == 15)
            def _():
                pltpu.sync_copy(
                    u_hbm.at[pl.ds(ubase + 16 * RPT, N - 16 * RPT)],
                    acc.at[pl.ds(16 * RPT, N - 16 * RPT)])
            plsc.subcore_barrier()

            def chunk_body(ch, _):
                ebase = pl.multiple_of(sid * EPT + ch * CE, 8)
                pltpu.sync_copy(esrc_hbm.at[pl.ds(ebase, CE)],
                                esrc_v.at[pl.ds(0, CE)])
                pltpu.sync_copy(edst_hbm.at[pl.ds(ebase, CE)],
                                edst_v.at[pl.ds(0, CE)])

                def build(k, gb_v, db_v):
                    for j in range(G // 16):
                        off = k * G + j * 16
                        gb_v[pl.ds(j * 16, 16)] = \
                            esrc_v[pl.ds(off, 16)] + ubase
                        db_v[pl.ds(j * 16, 16)] = edst_v[pl.ds(off, 16)]

                def pair_body(m, _):
                    k0 = 2 * m

                    build(k0, gb0_v, db0_v)
                    pltpu.async_copy(u_hbm.at[gb0_v], rb0, sg0)
                    build(k0 + 1, gb1_v, db1_v)
                    pltpu.async_copy(u_hbm.at[gb1_v], rb1, sg1)
                    pltpu.make_async_copy(u_hbm.at[gb0_v], rb0, sg0).wait()
                    pltpu.async_copy(rb0, acc.at[db0_v], ss0, add=True)
                    pltpu.make_async_copy(u_hbm.at[gb1_v], rb1, sg1).wait()
                    pltpu.async_copy(rb1, acc.at[db1_v], ss1, add=True)
                    pltpu.make_async_copy(rb0, acc.at[db0_v], ss0).wait()
                    pltpu.make_async_copy(rb1, acc.at[db1_v], ss1).wait()
                    return 0

                lax.fori_loop(0, CH // G // 2, pair_body, 0)
                return 0

            lax.fori_loop(0, NCH, chunk_body, 0)
            plsc.subcore_barrier()
            pltpu.sync_copy(acc.at[pl.ds(sid * RPT, RPT)],
                            y_hbm.at[pl.ds(ubase + sid * RPT, RPT)])

            @pl.when(sid == 15)
            def _():
                pltpu.sync_copy(
                    acc.at[pl.ds(16 * RPT, N - 16 * RPT)],
                    y_hbm.at[pl.ds(ubase + 16 * RPT, N - 16 * RPT)])
            plsc.subcore_barrier()
            return 0

        lax.fori_loop(0, NSLICE // 2, slice_body, 0)

    return agg


def _fin_kernel(B, N, P, OUT, BN):
    grid = (B, N // BN)
    NB = N // BN

    def body(y0, y1, y2r, y3, y4, y5, degp_ref, bz_ref, lwz_ref, lbz_ref,
             bh_ref, lwh_ref, lbh_ref, att_ref, wout_ref, bout_ref, o_ref):
        ys = (y0, y1, y2r, y3, y4, y5)
        deg = jnp.sum(degp_ref[...], axis=1) + 1.0
        dinv = lax.rsqrt(deg)
        c_z = jnp.dot(bz_ref[...], lwz_ref[:OUT, :],
                      preferred_element_type=jnp.float32) + lbz_ref[...]
        c_h = jnp.dot(bh_ref[...], lwh_ref[:OUT, :],
                      preferred_element_type=jnp.float32) + lbh_ref[...]
        a = att_ref[...]
        ea = jnp.exp(a - jnp.max(a))
        probs = ea / jnp.sum(ea)
        acc = jnp.zeros((BN, OUT), jnp.float32)
        for t in range(P):
            yslice = ys[t // 2][...]
            o = (t % 2) * 2 * OUT
            sz = yslice[:, o:o + OUT] * dinv[:, None] + c_z
            sh = yslice[:, o + OUT:o + 2 * OUT] * dinv[:, None] + c_h
            z = jax.nn.sigmoid(sz)
            ht = jnp.tanh(sh)
            acc = acc + probs[0, t] * (1.0 - z) * ht
        h = jax.nn.relu(acc)
        o_ref[0] = jnp.dot(h, wout_ref[...],
                           preferred_element_type=jnp.float32) + bout_ref[...]

    y_specs = [
        pl.BlockSpec((BN, 128),
                     lambda b, nb, si=si: ((b * (P // 2) + si) * NB + nb, 0))
        for si in range(P // 2)
    ]
    return pl.pallas_call(
        body,
        grid=grid,
        in_specs=y_specs + [
            pl.BlockSpec((BN, 32), lambda b, nb: (nb, 0)),
            pl.BlockSpec((1, OUT), lambda b, nb: (0, 0)),
            pl.BlockSpec((2 * OUT, OUT), lambda b, nb: (0, 0)),
            pl.BlockSpec((1, OUT), lambda b, nb: (0, 0)),
            pl.BlockSpec((1, OUT), lambda b, nb: (0, 0)),
            pl.BlockSpec((2 * OUT, OUT), lambda b, nb: (0, 0)),
            pl.BlockSpec((1, OUT), lambda b, nb: (0, 0)),
            pl.BlockSpec((1, P), lambda b, nb: (0, 0)),
            pl.BlockSpec((OUT, P), lambda b, nb: (0, 0)),
            pl.BlockSpec((1, P), lambda b, nb: (0, 0)),
        ],
        out_specs=pl.BlockSpec((1, BN, P), lambda b, nb: (b, nb, 0)),
        out_shape=jax.ShapeDtypeStruct((B, N, P), jnp.float32),
    )


def kernel(x, edge_index, W_z, b_z, Lw_z, Lb_z, W_r, b_r, Lw_r, Lb_r,
           W_h, b_h, Lw_h, Lb_h, att, W_out, b_out):
    B, N, F, P = x.shape
    E = edge_index.shape[1]
    OUT = W_z.shape[1]
    NSLICE = B * P * 2 * OUT // 128
    assert E % (16 * 2000) == 0 and N == 10000 and NSLICE % 2 == 0

    src = edge_index[0]
    dst = edge_index[1]
    x_t = jnp.transpose(x, (0, 3, 1, 2))

    deg_p = _deg_kernel(E, N)(dst).reshape(32, N).T
    u2 = _proj_kernel(B, N, F, P, OUT, 1000)(
        x_t, W_z, Lw_z, W_h, Lw_h, deg_p)
    y2 = _agg_kernel(E, N, NSLICE)(u2, src, dst)
    fin_in = [y2] * 6 + [
        deg_p, b_z.reshape(1, OUT), Lw_z, Lb_z.reshape(1, OUT),
        b_h.reshape(1, OUT), Lw_h, Lb_h.reshape(1, OUT),
        att.reshape(1, P), W_out, b_out.reshape(1, P)]
    return _fin_kernel(B, N, P, OUT, 1000)(*fin_in)


# speedup vs baseline: 1.0964x; 1.0964x over previous
import functools

import jax
import jax.numpy as jnp
from jax import lax
from jax.experimental import pallas as pl
from jax.experimental.pallas import tpu as pltpu
from jax.experimental.pallas import tpu_sc as plsc


def _deg_kernel(E, N):
    EPT = E // 32
    HALF = N // 2
    HPAD = HALF + 8
    NV = EPT // 16

    mesh = plsc.VectorSubcoreMesh(core_axis_name="c", subcore_axis_name="s")

    @functools.partial(
        pl.kernel,
        out_type=jax.ShapeDtypeStruct((32 * N,), jnp.float32),
        mesh=mesh,
        compiler_params=pltpu.CompilerParams(needs_layout_passes=False),
        scratch_types=[
            pltpu.VMEM((EPT,), jnp.int32),
            pltpu.VMEM((16 * HPAD,), jnp.float32),
            pltpu.VMEM((HPAD,), jnp.float32),
        ],
    )
    def deg(dst_hbm, out_hbm, dst_v, hist_v, acc_v):
        cid = lax.axis_index("c")
        sid = lax.axis_index("s")
        wid = sid * 2 + cid
        pltpu.sync_copy(dst_hbm.at[pl.ds(wid * EPT, EPT)], dst_v)
        lane_base = lax.iota(jnp.int32, 16) * HPAD
        ones = jnp.ones((16,), jnp.float32)
        zeros16 = jnp.zeros((16,), jnp.float32)
        for h in range(2):
            lo = h * HALF

            def zbody(j, _):
                hist_v[pl.ds(j * 16, 16)] = zeros16
                return 0
            lax.fori_loop(0, 16 * HPAD // 16, zbody, 0)

            def sbody(j, _):
                d = dst_v[pl.ds(j * 16, 16)]
                m = (d >= lo) & (d < lo + HALF)
                idx = lane_base + (d - lo)
                cur = plsc.load_gather(hist_v, [idx], mask=m)
                plsc.store_scatter(hist_v, [idx], cur + ones, mask=m)
                return 0
            lax.fori_loop(0, NV, sbody, 0)

            def rbody(j, _):
                a = hist_v[pl.ds(j * 16, 16)]
                for l in range(1, 16):
                    a = a + hist_v[pl.ds(l * HPAD + j * 16, 16)]
                acc_v[pl.ds(j * 16, 16)] = a
                return 0
            lax.fori_loop(0, HPAD // 16, rbody, 0)
            pltpu.sync_copy(acc_v.at[pl.ds(0, HALF)],
                            out_hbm.at[pl.ds(wid * N + lo, HALF)])

    return deg


def _proj_kernel(B, N, F, P, OUT, BN):
    grid = (B, P // 2, N // BN)
    NB = N // BN

    def body(x_ref, wz_ref, lwz_ref, wh_ref, lwh_ref, degp_ref, u_ref):
        wc = jnp.concatenate(
            [jnp.dot(wz_ref[...], lwz_ref[:OUT, :],
                     preferred_element_type=jnp.float32),
             jnp.dot(wh_ref[...], lwh_ref[:OUT, :],
                     preferred_element_type=jnp.float32)], axis=1)
        deg = jnp.sum(degp_ref[...], axis=1) + 1.0
        dinv = lax.rsqrt(deg)
        cols = []
        for tt in range(2):
            xt = x_ref[0, tt]
            m = jnp.dot(xt, wc, preferred_element_type=jnp.float32)
            cols.append(m * dinv[:, None])
        u_ref[...] = jnp.concatenate(cols, axis=1)

    return pl.pallas_call(
        body,
        grid=grid,
        in_specs=[
            pl.BlockSpec((1, 2, BN, F), lambda b, tp, nb: (b, tp, nb, 0)),
            pl.BlockSpec((F, OUT), lambda b, tp, nb: (0, 0)),
            pl.BlockSpec((2 * OUT, OUT), lambda b, tp, nb: (0, 0)),
            pl.BlockSpec((F, OUT), lambda b, tp, nb: (0, 0)),
            pl.BlockSpec((2 * OUT, OUT), lambda b, tp, nb: (0, 0)),
            pl.BlockSpec((BN, 32), lambda b, tp, nb: (nb, 0)),
        ],
        out_specs=pl.BlockSpec(
            (BN, 128), lambda b, tp, nb: ((b * (P // 2) + tp) * NB + nb, 0)),
        out_shape=jax.ShapeDtypeStruct((2 * P * N, 128), jnp.float32),
    )


def _agg_kernel(E, N, NSLICE):
    EPT = E // 16
    CH = 2048
    CE = 2000
    NCH = EPT // CE
    G = 64
    NBUF = 4
    SINK = N
    RPT = 624

    mesh = plsc.VectorSubcoreMesh(core_axis_name="c", subcore_axis_name="s")

    @functools.partial(
        pl.kernel,
        out_type=jax.ShapeDtypeStruct((NSLICE * N, 128), jnp.float32),
        mesh=mesh,
        compiler_params=pltpu.CompilerParams(needs_layout_passes=False),
        scratch_types=(
            [pltpu.VMEM((CH,), jnp.int32),
             pltpu.VMEM((CH,), jnp.int32)]
            + [pltpu.VMEM((G,), jnp.int32) for _ in range(2 * NBUF)]
            + [pltpu.VMEM((G, 128), jnp.float32) for _ in range(NBUF)]
            + [pltpu.VMEM_SHARED((N + 8, 128), jnp.float32)]
            + [pltpu.SemaphoreType.DMA for _ in range(2 * NBUF)]
        ),
    )
    def agg(u_hbm, esrc_hbm, edst_hbm, y_hbm, esrc_v, edst_v, *rest):
        gb = rest[0:NBUF]
        db = rest[NBUF:2 * NBUF]
        rb = rest[2 * NBUF:3 * NBUF]
        acc = rest[3 * NBUF]
        sg = rest[3 * NBUF + 1:3 * NBUF + 1 + NBUF]
        ss = rest[3 * NBUF + 1 + NBUF:3 * NBUF + 1 + 2 * NBUF]
        cid = lax.axis_index("c")
        sid = lax.axis_index("s")
        zero16 = jnp.zeros((16,), jnp.int32)
        sink16 = jnp.full((16,), SINK, jnp.int32)
        for q in range((CH - CE) // 16):
            esrc_v[pl.ds(CE + q * 16, 16)] = zero16
            edst_v[pl.ds(CE + q * 16, 16)] = sink16

        def slice_body(s_i, _):
            s = cid * (NSLICE // 2) + s_i
            ubase = pl.multiple_of(s * N, 8)
            pltpu.sync_copy(u_hbm.at[pl.ds(ubase + sid * RPT, RPT)],
                            acc.at[pl.ds(sid * RPT, RPT)])

            @pl.when(sid == 15)
            def _():
                pltpu.sync_copy(
                    u_hbm.at[pl.ds(ubase + 16 * RPT, N - 16 * RPT)],
                    acc.at[pl.ds(16 * RPT, N - 16 * RPT)])
            plsc.subcore_barrier()

            def chunk_body(ch, _):
                ebase = pl.multiple_of(sid * EPT + ch * CE, 8)
                pltpu.sync_copy(esrc_hbm.at[pl.ds(ebase, CE)],
                                esrc_v.at[pl.ds(0, CE)])
                pltpu.sync_copy(edst_hbm.at[pl.ds(ebase, CE)],
                                edst_v.at[pl.ds(0, CE)])

                def build(k, gb_v, db_v):
                    for j in range(G // 16):
                        off = k * G + j * 16
                        gb_v[pl.ds(j * 16, 16)] = \
                            esrc_v[pl.ds(off, 16)] + ubase
                        db_v[pl.ds(j * 16, 16)] = edst_v[pl.ds(off, 16)]

                def ring_body(m, _):
                    k0 = NBUF * m
                    for b in range(NBUF):
                        build(k0 + b, gb[b], db[b])
                        pltpu.async_copy(u_hbm.at[gb[b]], rb[b], sg[b])
                    for b in range(NBUF):
                        pltpu.make_async_copy(u_hbm.at[gb[b]], rb[b],
                                              sg[b]).wait()
                        pltpu.async_copy(rb[b], acc.at[db[b]], ss[b],
                                         add=True)
                    for b in range(NBUF):
                        pltpu.make_async_copy(rb[b], acc.at[db[b]],
                                              ss[b]).wait()
                    return 0

                lax.fori_loop(0, CH // G // NBUF, ring_body, 0)
                return 0

            lax.fori_loop(0, NCH, chunk_body, 0)
            plsc.subcore_barrier()
            pltpu.sync_copy(acc.at[pl.ds(sid * RPT, RPT)],
                            y_hbm.at[pl.ds(ubase + sid * RPT, RPT)])

            @pl.when(sid == 15)
            def _():
                pltpu.sync_copy(
                    acc.at[pl.ds(16 * RPT, N - 16 * RPT)],
                    y_hbm.at[pl.ds(ubase + 16 * RPT, N - 16 * RPT)])
            plsc.subcore_barrier()
            return 0

        lax.fori_loop(0, NSLICE // 2, slice_body, 0)

    return agg


def _fin_kernel(B, N, P, OUT, BN):
    grid = (B, N // BN)
    NB = N // BN

    def body(y0, y1, y2r, y3, y4, y5, degp_ref, bz_ref, lwz_ref, lbz_ref,
             bh_ref, lwh_ref, lbh_ref, att_ref, wout_ref, bout_ref, o_ref):
        ys = (y0, y1, y2r, y3, y4, y5)
        deg = jnp.sum(degp_ref[...], axis=1) + 1.0
        dinv = lax.rsqrt(deg)
        c_z = jnp.dot(bz_ref[...], lwz_ref[:OUT, :],
                      preferred_element_type=jnp.float32) + lbz_ref[...]
        c_h = jnp.dot(bh_ref[...], lwh_ref[:OUT, :],
                      preferred_element_type=jnp.float32) + lbh_ref[...]
        a = att_ref[...]
        ea = jnp.exp(a - jnp.max(a))
        probs = ea / jnp.sum(ea)
        acc = jnp.zeros((BN, OUT), jnp.float32)
        for t in range(P):
            yslice = ys[t // 2][...]
            o = (t % 2) * 2 * OUT
            sz = yslice[:, o:o + OUT] * dinv[:, None] + c_z
            sh = yslice[:, o + OUT:o + 2 * OUT] * dinv[:, None] + c_h
            z = jax.nn.sigmoid(sz)
            ht = jnp.tanh(sh)
            acc = acc + probs[0, t] * (1.0 - z) * ht
        h = jax.nn.relu(acc)
        o_ref[0] = jnp.dot(h, wout_ref[...],
                           preferred_element_type=jnp.float32) + bout_ref[...]

    y_specs = [
        pl.BlockSpec((BN, 128),
                     lambda b, nb, si=si: ((b * (P // 2) + si) * NB + nb, 0))
        for si in range(P // 2)
    ]
    return pl.pallas_call(
        body,
        grid=grid,
        in_specs=y_specs + [
            pl.BlockSpec((BN, 32), lambda b, nb: (nb, 0)),
            pl.BlockSpec((1, OUT), lambda b, nb: (0, 0)),
            pl.BlockSpec((2 * OUT, OUT), lambda b, nb: (0, 0)),
            pl.BlockSpec((1, OUT), lambda b, nb: (0, 0)),
            pl.BlockSpec((1, OUT), lambda b, nb: (0, 0)),
            pl.BlockSpec((2 * OUT, OUT), lambda b, nb: (0, 0)),
            pl.BlockSpec((1, OUT), lambda b, nb: (0, 0)),
            pl.BlockSpec((1, P), lambda b, nb: (0, 0)),
            pl.BlockSpec((OUT, P), lambda b, nb: (0, 0)),
            pl.BlockSpec((1, P), lambda b, nb: (0, 0)),
        ],
        out_specs=pl.BlockSpec((1, BN, P), lambda b, nb: (b, nb, 0)),
        out_shape=jax.ShapeDtypeStruct((B, N, P), jnp.float32),
    )


def kernel(x, edge_index, W_z, b_z, Lw_z, Lb_z, W_r, b_r, Lw_r, Lb_r,
           W_h, b_h, Lw_h, Lb_h, att, W_out, b_out):
    B, N, F, P = x.shape
    E = edge_index.shape[1]
    OUT = W_z.shape[1]
    NSLICE = B * P * 2 * OUT // 128
    assert E % (16 * 2000) == 0 and N == 10000 and NSLICE % 2 == 0

    src = edge_index[0]
    dst = edge_index[1]
    x_t = jnp.transpose(x, (0, 3, 1, 2))

    deg_p = _deg_kernel(E, N)(dst).reshape(32, N).T
    u2 = _proj_kernel(B, N, F, P, OUT, 1000)(
        x_t, W_z, Lw_z, W_h, Lw_h, deg_p)
    y2 = _agg_kernel(E, N, NSLICE)(u2, src, dst)
    fin_in = [y2] * 6 + [
        deg_p, b_z.reshape(1, OUT), Lw_z, Lb_z.reshape(1, OUT),
        b_h.reshape(1, OUT), Lw_h, Lb_h.reshape(1, OUT),
        att.reshape(1, P), W_out, b_out.reshape(1, P)]
    return _fin_kernel(B, N, P, OUT, 1000)(*fin_in)
